# baseline (speedup 1.0000x reference)
import functools

import jax
import jax.numpy as jnp
from jax.experimental import pallas as pl

B, C, N = 4, 64, 65536
HID, OUT = 32, 8
TOPK = 1024
TILE = 4096


def _mlp_body(f_ref, w1_ref, b1_ref, w2_ref, b2_ref, t_ref, s_ref):
    f = f_ref[0]
    h = jax.lax.dot_general(f, w1_ref[...], (((0,), (0,)), ((), ())),
                            preferred_element_type=jnp.float32)
    h = jnp.maximum(h + b1_ref[...], 0.0)
    t = jax.lax.dot_general(h, w2_ref[...], (((1,), (0,)), ((), ())),
                            preferred_element_type=jnp.float32)
    t = t + b2_ref[...]
    t_ref[0] = jnp.concatenate([t, jnp.zeros_like(t)], axis=1)
    s_ref[0, 0, :] = t[:, OUT - 1]


@functools.partial(jax.jit, static_argnums=())
def _mlp(features, W1, b1, W2, b2):
    grid = (B, N // TILE)
    t_pad, score = pl.pallas_call(
        _mlp_body,
        grid=grid,
        in_specs=[
            pl.BlockSpec((1, C, TILE), lambda b, n: (b, 0, n)),
            pl.BlockSpec((C, HID), lambda b, n: (0, 0)),
            pl.BlockSpec((1, HID), lambda b, n: (0, 0)),
            pl.BlockSpec((HID, OUT), lambda b, n: (0, 0)),
            pl.BlockSpec((1, OUT), lambda b, n: (0, 0)),
        ],
        out_specs=[
            pl.BlockSpec((1, TILE, 2 * OUT), lambda b, n: (b, n, 0)),
            pl.BlockSpec((1, 1, TILE), lambda b, n: (b, 0, n)),
        ],
        out_shape=[
            jax.ShapeDtypeStruct((B, N, 2 * OUT), jnp.float32),
            jax.ShapeDtypeStruct((B, 1, N), jnp.float32),
        ],
    )(features, W1, b1.reshape(1, HID), W2, b2.reshape(1, OUT))
    return t_pad, score


def kernel(points, features, W1, b1, W2, b2):
    t_pad, score = _mlp(features, W1, b1, W2, b2)
    _, idx = jax.lax.top_k(score[:, 0, :], TOPK)
    idx3 = jnp.broadcast_to(idx[:, :, None], (B, TOPK, OUT))
    return jnp.take_along_axis(t_pad[:, :, :OUT], idx3, axis=1)


# speedup vs baseline: 1.5480x; 1.5480x over previous
import functools

import jax
import jax.numpy as jnp
from jax import lax
from jax.experimental import pallas as pl
from jax.experimental.pallas import tpu as pltpu
from jax.experimental.pallas import tpu_sc as plsc

B, C, N = 4, 64, 65536
HID, OUT = 32, 8
TOPK = 1024
TILE = 4096

NCORE, NSUB, L = 2, 16, 16
BPC = B // NCORE
CHUNK = N // NSUB
CAND = 2 * TOPK

MININT = -(2**31)
MAXPOS = 0x7FFFFFFF


def _mlp_body(f_ref, w1_ref, b1_ref, w2_ref, b2_ref, t_ref, k_ref):
    f = f_ref[0]
    h = lax.dot_general(f, w1_ref[...], (((0,), (0,)), ((), ())),
                        preferred_element_type=jnp.float32)
    h = jnp.maximum(h + b1_ref[...], 0.0)
    t = lax.dot_general(h, w2_ref[...], (((1,), (0,)), ((), ())),
                        preferred_element_type=jnp.float32)
    t = t + b2_ref[...]
    t_ref[0] = jnp.concatenate([t, jnp.zeros_like(t)], axis=1)
    bits = lax.bitcast_convert_type(t[:, OUT - 1], jnp.int32)
    key = bits ^ (lax.shift_right_arithmetic(bits, 31) & jnp.int32(MAXPOS))
    k_ref[0, 0, :] = key


def _mlp(features, W1, b1, W2, b2):
    return pl.pallas_call(
        _mlp_body,
        grid=(B, N // TILE),
        in_specs=[
            pl.BlockSpec((1, C, TILE), lambda b, n: (b, 0, n)),
            pl.BlockSpec((C, HID), lambda b, n: (0, 0)),
            pl.BlockSpec((1, HID), lambda b, n: (0, 0)),
            pl.BlockSpec((HID, OUT), lambda b, n: (0, 0)),
            pl.BlockSpec((1, OUT), lambda b, n: (0, 0)),
        ],
        out_specs=[
            pl.BlockSpec((1, TILE, 2 * OUT), lambda b, n: (b, n, 0)),
            pl.BlockSpec((1, 1, TILE), lambda b, n: (b, 0, n)),
        ],
        out_shape=[
            jax.ShapeDtypeStruct((B, N, 2 * OUT), jnp.float32),
            jax.ShapeDtypeStruct((B, 1, N), jnp.int32),
        ],
    )(features, W1, b1.reshape(1, HID), W2, b2.reshape(1, OUT))


def _thresh_body(k_ref, t_ref):
    k = k_ref[0]

    def bit_round(bit, tb):
        trial = tb | (jnp.int32(1) << (31 - bit))
        trial_s = trial ^ jnp.int32(MININT)
        cnt = jnp.sum((k >= trial_s).astype(jnp.int32))
        return lax.select(cnt >= TOPK, trial, tb)

    tb = lax.fori_loop(0, 32, bit_round, jnp.int32(0))
    t_ref[0, 0, :] = jnp.full((L,), tb ^ jnp.int32(MININT), jnp.int32)


def _thresh(keys):
    return pl.pallas_call(
        _thresh_body,
        grid=(B,),
        in_specs=[pl.BlockSpec((1, 1, N), lambda b: (b, 0, 0))],
        out_specs=pl.BlockSpec((1, 1, L), lambda b: (b, 0, 0)),
        out_shape=jax.ShapeDtypeStruct((B, 1, L), jnp.int32),
    )(keys)


def _sc_body(keys_hbm, thr_hbm, tflat_hbm, oflat_hbm,
             keys_v, gtk_v, gti_v, eqi_v, gts_v, eqs_v,
             ck_v, cidx_v, rrow_v, tmp_v,
             counts_all_v, g2d_v, o2d_v, cols_v,
             counts_sh, candk_sh, candi_sh, sorted_sh, sem):
    c = lax.axis_index("c")
    s = lax.axis_index("s")
    iota = lax.iota(jnp.int32, L)

    for i in range(BPC):
        b = c * BPC + i
        base_n = b * N + s * CHUNK

        pltpu.sync_copy(keys_hbm.at[pl.ds(base_n, CHUNK)], keys_v)
        pltpu.sync_copy(thr_hbm.at[pl.ds(b * L, L)], tmp_v)
        t_splat = tmp_v[...]

        def compress(j, offs):
            gt_off, eq_off = offs
            k = keys_v[pl.ds(j * L, L)]
            idx = base_n + j * L + iota
            m_gt = k > t_splat
            m_eq = k == t_splat
            pg = plsc.cumsum(m_gt.astype(jnp.int32))
            tgt_g = gt_off + pg - 1
            plsc.store_scatter(gtk_v, [tgt_g], k, mask=m_gt)
            plsc.store_scatter(gti_v, [tgt_g], idx, mask=m_gt)
            pe = plsc.cumsum(m_eq.astype(jnp.int32))
            tgt_e = eq_off + pe - 1
            plsc.store_scatter(eqi_v, [tgt_e], idx, mask=m_eq)
            tmp_v[...] = plsc.all_reduce_population_count(m_gt)
            gt_off = gt_off + tmp_v[...][0]
            tmp_v[...] = plsc.all_reduce_population_count(m_eq)
            eq_off = eq_off + tmp_v[...][0]
            return (gt_off, eq_off)

        run_gt, run_eq = lax.fori_loop(
            0, CHUNK // L, compress, (jnp.int32(0), jnp.int32(0)))

        tmp_v[...] = jnp.full((L,), run_gt, jnp.int32)
        pltpu.sync_copy(tmp_v, counts_sh.at[s])
        tmp_v[...] = jnp.full((L,), run_eq, jnp.int32)
        pltpu.sync_copy(tmp_v, counts_sh.at[NSUB + s])
        plsc.subcore_barrier()
        pltpu.sync_copy(counts_sh, counts_all_v)

        gt_base = jnp.int32(0)
        eq_base = jnp.int32(0)
        total_gt = jnp.int32(0)
        for v in range(NSUB):
            gcnt = counts_all_v[v][0]
            ecnt = counts_all_v[NSUB + v][0]
            before = (v < s).astype(jnp.int32)
            gt_base = gt_base + gcnt * before
            eq_base = eq_base + ecnt * before
            total_gt = total_gt + gcnt
        eq_base = eq_base + total_gt

        def fill_gt(v, _):
            r = v // (128 // L)
            u = v % (128 // L)
            pos = v * L + iota
            slot = gt_base + pos
            slot = jnp.where(pos < run_gt, slot, TOPK + (slot & (TOPK - 1)))
            gts_v[r, pl.ds(u * L, L)] = slot
            return 0

        def fill_eq(v, _):
            r = v // (128 // L)
            u = v % (128 // L)
            pos = v * L + iota
            slot = eq_base + pos
            ok = (pos < run_eq) & (slot < TOPK)
            slot = jnp.where(ok, slot, TOPK + (slot & (TOPK - 1)))
            eqs_v[r, pl.ds(u * L, L)] = slot
            return 0

        lax.fori_loop(0, (TOPK // 128) * (128 // L), fill_gt, 0)
        lax.fori_loop(0, (4 * TOPK // 128) * (128 // L), fill_eq, 0)

        def scat_gt(r, _):
            pltpu.sync_copy(gtk_v.at[pl.ds(r * 128, 128)], candk_sh.at[gts_v.at[r]])
            pltpu.sync_copy(gti_v.at[pl.ds(r * 128, 128)], candi_sh.at[gts_v.at[r]])
            return 0

        def scat_eq(r, _):
            pltpu.sync_copy(eqi_v.at[pl.ds(r * 128, 128)], candi_sh.at[eqs_v.at[r]])
            return 0

        lax.fori_loop(0, (run_gt + 127) // 128, scat_gt, 0)
        lax.fori_loop(0, (run_eq + 127) // 128, scat_eq, 0)
        plsc.subcore_barrier()

        pltpu.sync_copy(candk_sh.at[pl.ds(0, TOPK)], ck_v)
        pltpu.sync_copy(candi_sh.at[pl.ds(s * (TOPK // NSUB), TOPK // NSUB)],
                        cidx_v)

        def patch(j, _):
            pos = j * L + iota
            v = ck_v[pl.ds(j * L, L)]
            ck_v[pl.ds(j * L, L)] = jnp.where(pos < total_gt, v, t_splat)
            return 0

        lax.fori_loop(0, TOPK // L, patch, 0)

        for i_blk in range(TOPK // NSUB // L):
            my0 = s * (TOPK // NSUB) + i_blk * L
            ki = ck_v[pl.ds(my0, L)]
            pos_i = my0 + iota

            def jbody(j, rank):
                jbase = j * L
                for r in range(L):
                    gidx = jbase + ((iota + r) & (L - 1))
                    kjr = plsc.load_gather(ck_v, [gidx])
                    gt_hit = (kjr > ki) | ((kjr == ki) & (gidx < pos_i))
                    rank = rank + gt_hit.astype(jnp.int32)
                return rank

            rank = lax.fori_loop(0, TOPK // L, jbody,
                                 jnp.zeros((L,), jnp.int32))
            rrow_v[pl.ds(i_blk * L, L)] = rank

        for v in range(TOPK // NSUB // L):
            idxv = cidx_v[pl.ds(v * L, L)]
            rnkv = rrow_v[pl.ds(v * L, L)]
            for j in range(2 * OUT):
                g2d_v[j, pl.ds(v * L, L)] = idxv * (2 * OUT) + j
                o2d_v[j, pl.ds(v * L, L)] = rnkv * (2 * OUT) + j
        copies = [
            pltpu.async_copy(tflat_hbm.at[g2d_v.at[j]], cols_v.at[j], sem)
            for j in range(2 * OUT)
        ]
        for cp in copies:
            cp.wait()
        for j in range(2 * OUT):
            pltpu.sync_copy(cols_v.at[j], sorted_sh.at[o2d_v.at[j]])
        plsc.subcore_barrier()
        seg = 2 * OUT * TOPK // NSUB
        pltpu.sync_copy(
            sorted_sh.at[pl.ds(s * seg, seg)],
            oflat_hbm.at[pl.ds(b * 2 * OUT * TOPK + s * seg, seg)])
        plsc.subcore_barrier()


def _sc_select(keys_flat, thr_flat, t_flat):
    kern = pl.kernel(
        _sc_body,
        out_type=jax.ShapeDtypeStruct((B * TOPK * 2 * OUT,), jnp.float32),
        mesh=plsc.VectorSubcoreMesh(core_axis_name="c", subcore_axis_name="s",
                                    num_cores=NCORE, num_subcores=NSUB),
        compiler_params=pltpu.CompilerParams(needs_layout_passes=False),
        scratch_types=[
            pltpu.VMEM((CHUNK,), jnp.int32),
            pltpu.VMEM((TOPK + L,), jnp.int32),
            pltpu.VMEM((TOPK + L,), jnp.int32),
            pltpu.VMEM((4 * TOPK + L,), jnp.int32),
            pltpu.VMEM((TOPK // 128, 128), jnp.int32),
            pltpu.VMEM((4 * TOPK // 128, 128), jnp.int32),
            pltpu.VMEM((TOPK,), jnp.int32),
            pltpu.VMEM((TOPK // NSUB,), jnp.int32),
            pltpu.VMEM((TOPK // NSUB,), jnp.int32),
            pltpu.VMEM((L,), jnp.int32),
            pltpu.VMEM((2 * NSUB, L), jnp.int32),
            pltpu.VMEM((2 * OUT, TOPK // NSUB), jnp.int32),
            pltpu.VMEM((2 * OUT, TOPK // NSUB), jnp.int32),
            pltpu.VMEM((2 * OUT, TOPK // NSUB), jnp.float32),
            pltpu.VMEM_SHARED((2 * NSUB, L), jnp.int32),
            pltpu.VMEM_SHARED((CAND,), jnp.int32),
            pltpu.VMEM_SHARED((CAND,), jnp.int32),
            pltpu.VMEM_SHARED((2 * OUT * TOPK,), jnp.float32),
            pltpu.SemaphoreType.DMA,
        ],
    )
    return kern(keys_flat, thr_flat, t_flat)


def kernel(points, features, W1, b1, W2, b2):
    t_pad, keys = _mlp(features, W1, b1, W2, b2)
    thr = _thresh(keys)
    out_flat = _sc_select(keys.reshape(B * N), thr.reshape(B * L),
                          t_pad.reshape(B * N * 2 * OUT))
    return out_flat.reshape(B, TOPK, 2 * OUT)[:, :, :OUT]


# speedup vs baseline: 3.2992x; 2.1313x over previous
import functools

import jax
import jax.numpy as jnp
from jax import lax
from jax.experimental import pallas as pl
from jax.experimental.pallas import tpu as pltpu
from jax.experimental.pallas import tpu_sc as plsc

B, C, N = 4, 64, 65536
HID, OUT = 32, 8
TOPK = 1024
TILE = 4096

NCORE, NSUB, L = 2, 16, 16
BPC = B // NCORE
CHUNK = N // NSUB
CAND = 2 * TOPK

MININT = -(2**31)
MAXPOS = 0x7FFFFFFF


def _mlp_body(f_ref, w1_ref, b1_ref, w2_ref, b2_ref, t_ref, k_ref):
    f = f_ref[0]
    h = lax.dot_general(w1_ref[...], f, (((0,), (0,)), ((), ())),
                        preferred_element_type=jnp.float32)
    h = jnp.maximum(h + b1_ref[...], 0.0)
    t = lax.dot_general(w2_ref[...], h, (((0,), (0,)), ((), ())),
                        preferred_element_type=jnp.float32)
    t = t + b2_ref[...]
    t_ref[0] = t
    bits = lax.bitcast_convert_type(t[OUT - 1:OUT, :], jnp.int32)
    key = bits ^ (lax.shift_right_arithmetic(bits, 31) & jnp.int32(MAXPOS))
    k_ref[0] = key


def _mlp(features, W1, b1, W2, b2):
    return pl.pallas_call(
        _mlp_body,
        grid=(B, N // TILE),
        in_specs=[
            pl.BlockSpec((1, C, TILE), lambda b, n: (b, 0, n)),
            pl.BlockSpec((C, HID), lambda b, n: (0, 0)),
            pl.BlockSpec((HID, 1), lambda b, n: (0, 0)),
            pl.BlockSpec((HID, OUT), lambda b, n: (0, 0)),
            pl.BlockSpec((OUT, 1), lambda b, n: (0, 0)),
        ],
        out_specs=[
            pl.BlockSpec((1, OUT, TILE), lambda b, n: (b, 0, n)),
            pl.BlockSpec((1, 1, TILE), lambda b, n: (b, 0, n)),
        ],
        out_shape=[
            jax.ShapeDtypeStruct((B, OUT, N), jnp.float32),
            jax.ShapeDtypeStruct((B, 1, N), jnp.int32),
        ],
    )(features, W1, b1.reshape(HID, 1), W2, b2.reshape(OUT, 1))


def _thresh_body(k_ref, t_ref):
    k = k_ref[...]

    def bit_round(bit, tb):
        trial = tb | (jnp.int32(1) << (31 - bit))
        trial_s = trial ^ jnp.int32(MININT)
        cnt = jnp.sum((k >= trial_s).astype(jnp.int32), axis=2, keepdims=True)
        return jnp.where(cnt >= TOPK, trial, tb)

    tb = lax.fori_loop(0, 32, bit_round, jnp.zeros((B, 1, 1), jnp.int32))
    t_ref[...] = jnp.broadcast_to(tb ^ jnp.int32(MININT), (B, 1, L))


def _thresh(keys):
    return pl.pallas_call(
        _thresh_body,
        grid=(1,),
        in_specs=[pl.BlockSpec((B, 1, N), lambda g: (0, 0, 0))],
        out_specs=pl.BlockSpec((B, 1, L), lambda g: (0, 0, 0)),
        out_shape=jax.ShapeDtypeStruct((B, 1, L), jnp.int32),
    )(keys)


def _sc_body(keys_hbm, thr_hbm, tflat_hbm, oflat_hbm,
             keys_v, gtk_v, gti_v, eqi_v, gts_v, eqs_v,
             ck_v, cidx_v, rrow_v, tmp_v,
             counts_all_v, g2d_v, o2d_v, cols_v,
             counts_sh, candk_sh, candi_sh, sorted_sh, sem):
    c = lax.axis_index("c")
    s = lax.axis_index("s")
    iota = lax.iota(jnp.int32, L)

    for i in range(BPC):
        b = c * BPC + i
        base_n = b * N + s * CHUNK

        pltpu.sync_copy(keys_hbm.at[pl.ds(base_n, CHUNK)], keys_v)
        pltpu.sync_copy(thr_hbm.at[pl.ds(b * L, L)], tmp_v)
        t_splat = tmp_v[...]

        def compress(j, offs):
            gt_off, eq_off = offs
            k = keys_v[pl.ds(j * L, L)]
            idx = s * CHUNK + j * L + iota
            m_gt = k > t_splat
            m_eq = k == t_splat
            pg = plsc.cumsum(m_gt.astype(jnp.int32))
            tgt_g = gt_off + pg - 1
            plsc.store_scatter(gtk_v, [tgt_g], k, mask=m_gt)
            plsc.store_scatter(gti_v, [tgt_g], idx, mask=m_gt)
            pe = plsc.cumsum(m_eq.astype(jnp.int32))
            tgt_e = eq_off + pe - 1
            plsc.store_scatter(eqi_v, [tgt_e], idx, mask=m_eq)
            tmp_v[...] = plsc.all_reduce_population_count(m_gt)
            gt_off = gt_off + tmp_v[...][0]
            tmp_v[...] = plsc.all_reduce_population_count(m_eq)
            eq_off = eq_off + tmp_v[...][0]
            return (gt_off, eq_off)

        run_gt, run_eq = lax.fori_loop(
            0, CHUNK // L, compress, (jnp.int32(0), jnp.int32(0)))

        tmp_v[...] = jnp.full((L,), run_gt, jnp.int32)
        pltpu.sync_copy(tmp_v, counts_sh.at[s])
        tmp_v[...] = jnp.full((L,), run_eq, jnp.int32)
        pltpu.sync_copy(tmp_v, counts_sh.at[NSUB + s])
        plsc.subcore_barrier()
        pltpu.sync_copy(counts_sh, counts_all_v)

        gt_base = jnp.int32(0)
        eq_base = jnp.int32(0)
        total_gt = jnp.int32(0)
        for v in range(NSUB):
            gcnt = counts_all_v[v][0]
            ecnt = counts_all_v[NSUB + v][0]
            before = (v < s).astype(jnp.int32)
            gt_base = gt_base + gcnt * before
            eq_base = eq_base + ecnt * before
            total_gt = total_gt + gcnt
        eq_base = eq_base + total_gt

        def fill_gt(v, _):
            r = v // (128 // L)
            u = v % (128 // L)
            pos = v * L + iota
            slot = gt_base + pos
            slot = jnp.where(pos < run_gt, slot, TOPK + (slot & (TOPK - 1)))
            gts_v[r, pl.ds(u * L, L)] = slot
            return 0

        def fill_eq(v, _):
            r = v // (128 // L)
            u = v % (128 // L)
            pos = v * L + iota
            slot = eq_base + pos
            ok = (pos < run_eq) & (slot < TOPK)
            slot = jnp.where(ok, slot, TOPK + (slot & (TOPK - 1)))
            eqs_v[r, pl.ds(u * L, L)] = slot
            return 0

        lax.fori_loop(0, (TOPK // 128) * (128 // L), fill_gt, 0)
        lax.fori_loop(0, (4 * TOPK // 128) * (128 // L), fill_eq, 0)

        def scat_gt(r, _):
            pltpu.sync_copy(gtk_v.at[pl.ds(r * 128, 128)], candk_sh.at[gts_v.at[r]])
            pltpu.sync_copy(gti_v.at[pl.ds(r * 128, 128)], candi_sh.at[gts_v.at[r]])
            return 0

        def scat_eq(r, _):
            pltpu.sync_copy(eqi_v.at[pl.ds(r * 128, 128)], candi_sh.at[eqs_v.at[r]])
            return 0

        lax.fori_loop(0, (run_gt + 127) // 128, scat_gt, 0)
        lax.fori_loop(0, (run_eq + 127) // 128, scat_eq, 0)
        plsc.subcore_barrier()

        pltpu.sync_copy(candk_sh.at[pl.ds(0, TOPK)], ck_v)
        pltpu.sync_copy(candi_sh.at[pl.ds(s * (TOPK // NSUB), TOPK // NSUB)],
                        cidx_v)

        def patch(j, _):
            pos = j * L + iota
            v = ck_v[pl.ds(j * L, L)]
            ck_v[pl.ds(j * L, L)] = jnp.where(pos < total_gt, v, t_splat)
            return 0

        lax.fori_loop(0, TOPK // L, patch, 0)

        for i_blk in range(TOPK // NSUB // L):
            my0 = s * (TOPK // NSUB) + i_blk * L
            ki = ck_v[pl.ds(my0, L)]
            pos_i = my0 + iota

            def jbody(j, rank):
                jbase = j * L
                for r in range(L):
                    gidx = jbase + ((iota + r) & (L - 1))
                    kjr = plsc.load_gather(ck_v, [gidx])
                    gt_hit = (kjr > ki) | ((kjr == ki) & (gidx < pos_i))
                    rank = rank + gt_hit.astype(jnp.int32)
                return rank

            rank = lax.fori_loop(0, TOPK // L, jbody,
                                 jnp.zeros((L,), jnp.int32))
            rrow_v[pl.ds(i_blk * L, L)] = rank

        for v in range(TOPK // NSUB // L):
            idxv = cidx_v[pl.ds(v * L, L)]
            rnkv = rrow_v[pl.ds(v * L, L)]
            for j in range(OUT):
                g2d_v[j, pl.ds(v * L, L)] = (b * OUT + j) * N + idxv
                o2d_v[j, pl.ds(v * L, L)] = rnkv * OUT + j
        copies = [
            pltpu.async_copy(tflat_hbm.at[g2d_v.at[j]], cols_v.at[j], sem)
            for j in range(OUT)
        ]
        for cp in copies:
            cp.wait()
        for j in range(OUT):
            pltpu.sync_copy(cols_v.at[j], sorted_sh.at[o2d_v.at[j]])
        plsc.subcore_barrier()
        seg = OUT * TOPK // NSUB
        pltpu.sync_copy(
            sorted_sh.at[pl.ds(s * seg, seg)],
            oflat_hbm.at[pl.ds(b * OUT * TOPK + s * seg, seg)])
        plsc.subcore_barrier()


def _sc_select(keys_flat, thr_flat, t_flat):
    kern = pl.kernel(
        _sc_body,
        out_type=jax.ShapeDtypeStruct((B * TOPK * OUT,), jnp.float32),
        mesh=plsc.VectorSubcoreMesh(core_axis_name="c", subcore_axis_name="s",
                                    num_cores=NCORE, num_subcores=NSUB),
        compiler_params=pltpu.CompilerParams(needs_layout_passes=False),
        scratch_types=[
            pltpu.VMEM((CHUNK,), jnp.int32),
            pltpu.VMEM((TOPK + L,), jnp.int32),
            pltpu.VMEM((TOPK + L,), jnp.int32),
            pltpu.VMEM((4 * TOPK + L,), jnp.int32),
            pltpu.VMEM((TOPK // 128, 128), jnp.int32),
            pltpu.VMEM((4 * TOPK // 128, 128), jnp.int32),
            pltpu.VMEM((TOPK,), jnp.int32),
            pltpu.VMEM((TOPK // NSUB,), jnp.int32),
            pltpu.VMEM((TOPK // NSUB,), jnp.int32),
            pltpu.VMEM((L,), jnp.int32),
            pltpu.VMEM((2 * NSUB, L), jnp.int32),
            pltpu.VMEM((OUT, TOPK // NSUB), jnp.int32),
            pltpu.VMEM((OUT, TOPK // NSUB), jnp.int32),
            pltpu.VMEM((OUT, TOPK // NSUB), jnp.float32),
            pltpu.VMEM_SHARED((2 * NSUB, L), jnp.int32),
            pltpu.VMEM_SHARED((CAND,), jnp.int32),
            pltpu.VMEM_SHARED((CAND,), jnp.int32),
            pltpu.VMEM_SHARED((OUT * TOPK,), jnp.float32),
            pltpu.SemaphoreType.DMA,
        ],
    )
    return kern(keys_flat, thr_flat, t_flat)


def kernel(points, features, W1, b1, W2, b2):
    t_cn, keys = _mlp(features, W1, b1, W2, b2)
    thr = _thresh(keys)
    out_flat = _sc_select(keys.reshape(B * N), thr.reshape(B * L),
                          t_cn.reshape(B * OUT * N))
    return out_flat.reshape(B, TOPK, OUT)


# speedup vs baseline: 3.6537x; 1.1074x over previous
import jax
import jax.numpy as jnp
from jax import lax
from jax.experimental import pallas as pl
from jax.experimental.pallas import tpu as pltpu
from jax.experimental.pallas import tpu_sc as plsc

B, C, N = 4, 64, 65536
HID, OUT = 32, 8
TOPK = 1024
TILE = 8192

NCORE, NSUB, L = 2, 16, 16
BPC = B // NCORE
CHUNK = N // NSUB
CAND = 2 * TOPK
MYC = TOPK // NSUB

MININT = -(2**31)
MAXPOS = 0x7FFFFFFF


def _mlp_body(f_ref, w1_ref, b1_ref, w2_ref, b2_ref, *out_refs):
    f = f_ref[0]
    h = lax.dot_general(w1_ref[...], f, (((0,), (0,)), ((), ())),
                        preferred_element_type=jnp.float32)
    h = jnp.maximum(h + b1_ref[...], 0.0)
    t = lax.dot_general(w2_ref[...], h, (((0,), (0,)), ((), ())),
                        preferred_element_type=jnp.float32)
    t = t + b2_ref[...]
    for j in range(OUT):
        out_refs[j][...] = t[j]
    bits = lax.bitcast_convert_type(t[OUT - 1], jnp.int32)
    key = bits ^ (lax.shift_right_arithmetic(bits, 31) & jnp.int32(MAXPOS))
    out_refs[OUT][...] = key


def _mlp(features, W1, b1, W2, b2):
    flat = jax.ShapeDtypeStruct((B * N,), jnp.float32)
    return pl.pallas_call(
        _mlp_body,
        grid=(B, N // TILE),
        in_specs=[
            pl.BlockSpec((1, C, TILE), lambda b, n: (b, 0, n)),
            pl.BlockSpec((C, HID), lambda b, n: (0, 0)),
            pl.BlockSpec((HID, 1), lambda b, n: (0, 0)),
            pl.BlockSpec((HID, OUT), lambda b, n: (0, 0)),
            pl.BlockSpec((OUT, 1), lambda b, n: (0, 0)),
        ],
        out_specs=[
            pl.BlockSpec((TILE,), lambda b, n: (b * (N // TILE) + n,))
            for _ in range(OUT + 1)
        ],
        out_shape=[flat] * OUT + [jax.ShapeDtypeStruct((B * N,), jnp.int32)],
    )(features, W1, b1.reshape(HID, 1), W2, b2.reshape(OUT, 1))


def _thresh_body(k_ref, t_ref):
    for b in range(B):
        k = k_ref[pl.ds(b * N, N)]

        def bit_round(bit, tb):
            trial = tb | (jnp.int32(1) << (31 - bit))
            trial_s = trial ^ jnp.int32(MININT)
            cnt = jnp.sum((k >= trial_s).astype(jnp.int32))
            return lax.select(cnt >= TOPK, trial, tb)

        tb = lax.fori_loop(0, 32, bit_round, jnp.int32(0))
        t_ref[pl.ds(b * L, L)] = jnp.full((L,), tb ^ jnp.int32(MININT),
                                          jnp.int32)


def _thresh(keys):
    return pl.pallas_call(
        _thresh_body,
        grid=(1,),
        in_specs=[pl.BlockSpec((B * N,), lambda g: (0,))],
        out_specs=pl.BlockSpec((B * L,), lambda g: (0,)),
        out_shape=jax.ShapeDtypeStruct((B * L,), jnp.int32),
    )(keys)


def _sc_body(keys_hbm, thr_hbm, t0, t1, t2, t3, t4, t5, t6, t7, oflat_hbm,
             keys_v, gtk_v, gti_v, eqi_v, gts_v, eqs_v,
             ck_v, cidx_v, rrow_v, tmp_v,
             counts_all_v, g1d_v, o2d_v, cols_v,
             counts_sh, candk_sh, candi_sh, sorted_sh, sem):
    ts = (t0, t1, t2, t3, t4, t5, t6, t7)
    c = lax.axis_index("c")
    s = lax.axis_index("s")
    iota = lax.iota(jnp.int32, L)

    for i in range(BPC):
        b = c * BPC + i
        base_n = b * N + s * CHUNK

        pltpu.sync_copy(keys_hbm.at[pl.ds(base_n, CHUNK)], keys_v)
        pltpu.sync_copy(thr_hbm.at[pl.ds(b * L, L)], tmp_v)
        t_splat = tmp_v[...]

        def compress(j, offs):
            gt_off, eq_off = offs
            k = keys_v[pl.ds(j * L, L)]
            idx = s * CHUNK + j * L + iota
            m_gt = k > t_splat
            m_eq = k == t_splat
            pg = plsc.cumsum(m_gt.astype(jnp.int32))
            tgt_g = gt_off + pg - 1
            plsc.store_scatter(gtk_v, [tgt_g], k, mask=m_gt)
            plsc.store_scatter(gti_v, [tgt_g], idx, mask=m_gt)
            pe = plsc.cumsum(m_eq.astype(jnp.int32))
            tgt_e = eq_off + pe - 1
            plsc.store_scatter(eqi_v, [tgt_e], idx, mask=m_eq)
            tmp_v[...] = plsc.all_reduce_population_count(m_gt)
            gt_off = gt_off + tmp_v[...][0]
            tmp_v[...] = plsc.all_reduce_population_count(m_eq)
            eq_off = eq_off + tmp_v[...][0]
            return (gt_off, eq_off)

        run_gt, run_eq = lax.fori_loop(
            0, CHUNK // L, compress, (jnp.int32(0), jnp.int32(0)))

        tmp_v[...] = jnp.full((L,), run_gt, jnp.int32)
        pltpu.sync_copy(tmp_v, counts_sh.at[s])
        tmp_v[...] = jnp.full((L,), run_eq, jnp.int32)
        pltpu.sync_copy(tmp_v, counts_sh.at[NSUB + s])
        plsc.subcore_barrier()
        pltpu.sync_copy(counts_sh, counts_all_v)

        gt_base = jnp.int32(0)
        eq_base = jnp.int32(0)
        total_gt = jnp.int32(0)
        for v in range(NSUB):
            gcnt = counts_all_v[v][0]
            ecnt = counts_all_v[NSUB + v][0]
            before = (v < s).astype(jnp.int32)
            gt_base = gt_base + gcnt * before
            eq_base = eq_base + ecnt * before
            total_gt = total_gt + gcnt
        eq_base = eq_base + total_gt

        def fill_gt(v, _):
            r = v // (128 // L)
            u = v % (128 // L)
            pos = v * L + iota
            slot = gt_base + pos
            slot = jnp.where(pos < run_gt, slot, TOPK + (slot & (TOPK - 1)))
            gts_v[r, pl.ds(u * L, L)] = slot
            return 0

        def fill_eq(v, _):
            r = v // (128 // L)
            u = v % (128 // L)
            pos = v * L + iota
            slot = eq_base + pos
            ok = (pos < run_eq) & (slot < TOPK)
            slot = jnp.where(ok, slot, TOPK + (slot & (TOPK - 1)))
            eqs_v[r, pl.ds(u * L, L)] = slot
            return 0

        lax.fori_loop(0, (TOPK // 128) * (128 // L), fill_gt, 0)
        lax.fori_loop(0, (4 * TOPK // 128) * (128 // L), fill_eq, 0)

        def scat_gt(r, _):
            pltpu.sync_copy(gtk_v.at[pl.ds(r * 128, 128)],
                            candk_sh.at[gts_v.at[r]])
            pltpu.sync_copy(gti_v.at[pl.ds(r * 128, 128)],
                            candi_sh.at[gts_v.at[r]])
            return 0

        def scat_eq(r, _):
            pltpu.sync_copy(eqi_v.at[pl.ds(r * 128, 128)],
                            candi_sh.at[eqs_v.at[r]])
            return 0

        lax.fori_loop(0, (run_gt + 127) // 128, scat_gt, 0)
        lax.fori_loop(0, (run_eq + 127) // 128, scat_eq, 0)
        plsc.subcore_barrier()

        pltpu.sync_copy(candk_sh.at[pl.ds(0, TOPK)], ck_v)
        pltpu.sync_copy(candi_sh.at[pl.ds(s * MYC, MYC)], cidx_v)

        def patch(j, _):
            pos = j * L + iota
            vv = ck_v[pl.ds(j * L, L)]
            ck_v[pl.ds(j * L, L)] = jnp.where(pos < total_gt, vv, t_splat)
            return 0

        lax.fori_loop(0, TOPK // L, patch, 0)

        for i_blk in range(MYC // L):
            my0 = s * MYC + i_blk * L
            ki = ck_v[pl.ds(my0, L)]
            pos_i = my0 + iota

            def jbody(j, rank):
                jbase = j * L
                for r in range(L):
                    gidx = jbase + ((iota + r) & (L - 1))
                    kjr = plsc.load_gather(ck_v, [gidx])
                    hit = (kjr > ki) | ((kjr == ki) & (gidx < pos_i))
                    rank = rank + hit.astype(jnp.int32)
                return rank

            rank = lax.fori_loop(0, TOPK // L, jbody,
                                 jnp.zeros((L,), jnp.int32))
            rrow_v[pl.ds(i_blk * L, L)] = rank

        for v in range(MYC // L):
            idxv = cidx_v[pl.ds(v * L, L)]
            rnkv = rrow_v[pl.ds(v * L, L)]
            g1d_v[pl.ds(v * L, L)] = b * N + idxv
            for j in range(OUT):
                o2d_v[j, pl.ds(v * L, L)] = rnkv * OUT + j
        copies = [
            pltpu.async_copy(ts[j].at[g1d_v], cols_v.at[j], sem)
            for j in range(OUT)
        ]
        for cp in copies:
            cp.wait()
        for j in range(OUT):
            pltpu.sync_copy(cols_v.at[j], sorted_sh.at[o2d_v.at[j]])
        plsc.subcore_barrier()
        seg = OUT * TOPK // NSUB
        pltpu.sync_copy(
            sorted_sh.at[pl.ds(s * seg, seg)],
            oflat_hbm.at[pl.ds(b * OUT * TOPK + s * seg, seg)])
        plsc.subcore_barrier()


def _sc_select(keys_flat, thr_flat, t_chans):
    kern = pl.kernel(
        _sc_body,
        out_type=jax.ShapeDtypeStruct((B * TOPK * OUT,), jnp.float32),
        mesh=plsc.VectorSubcoreMesh(core_axis_name="c", subcore_axis_name="s",
                                    num_cores=NCORE, num_subcores=NSUB),
        compiler_params=pltpu.CompilerParams(needs_layout_passes=False),
        scratch_types=[
            pltpu.VMEM((CHUNK,), jnp.int32),
            pltpu.VMEM((TOPK + L,), jnp.int32),
            pltpu.VMEM((TOPK + L,), jnp.int32),
            pltpu.VMEM((4 * TOPK + L,), jnp.int32),
            pltpu.VMEM((TOPK // 128, 128), jnp.int32),
            pltpu.VMEM((4 * TOPK // 128, 128), jnp.int32),
            pltpu.VMEM((TOPK,), jnp.int32),
            pltpu.VMEM((MYC,), jnp.int32),
            pltpu.VMEM((MYC,), jnp.int32),
            pltpu.VMEM((L,), jnp.int32),
            pltpu.VMEM((2 * NSUB, L), jnp.int32),
            pltpu.VMEM((MYC,), jnp.int32),
            pltpu.VMEM((OUT, MYC), jnp.int32),
            pltpu.VMEM((OUT, MYC), jnp.float32),
            pltpu.VMEM_SHARED((2 * NSUB, L), jnp.int32),
            pltpu.VMEM_SHARED((CAND,), jnp.int32),
            pltpu.VMEM_SHARED((CAND,), jnp.int32),
            pltpu.VMEM_SHARED((OUT * TOPK,), jnp.float32),
            pltpu.SemaphoreType.DMA,
        ],
    )
    return kern(keys_flat, thr_flat, *t_chans)


def kernel(points, features, W1, b1, W2, b2):
    *t_chans, keys = _mlp(features, W1, b1, W2, b2)
    thr = _thresh(keys)
    out_flat = _sc_select(keys, thr, t_chans)
    return out_flat.reshape(B, TOPK, OUT)
